# baseline (speedup 1.0000x reference)
import functools

import jax
import jax.numpy as jnp
import numpy as np
from jax import lax
from jax.experimental import pallas as pl
from jax.experimental.pallas import tpu as pltpu
from jax.experimental.pallas import tpu_sc as plsc

T = 2048
H = 1024
NH = 16
NKV = 8
HD = 64
E = 8
TOPK = 2
FF = 512
EPS = 1e-6
THETA = 10000.0

BT = 256
BQ = 256
TILE = 128
PADMAX = TOPK * T + E * TILE
NT = PADMAX // TILE
NEG = -1e30

SC_NC = 2
SC_NS = 16
SC_NW = SC_NC * SC_NS


def _qkv_body(x_ref, cos_ref, sin_ref, wqkv_ref, ln1_ref, qw_ref, kw_ref,
              q_ref, k_ref, v_ref):
    x = x_ref[...]
    var = jnp.mean(x * x, axis=-1, keepdims=True)
    xn = x * lax.rsqrt(var + EPS) * ln1_ref[...]
    qkv = jnp.dot(xn, wqkv_ref[...], preferred_element_type=jnp.float32)
    cos = cos_ref[...]
    sin = sin_ref[...]

    def head_norm_rope(xh, w):
        ss = jnp.mean(xh * xh, axis=-1, keepdims=True)
        xh = xh * lax.rsqrt(ss + EPS) * w
        x1 = xh[:, :HD // 2]
        x2 = xh[:, HD // 2:]
        return jnp.concatenate([x1 * cos - x2 * sin, x2 * cos + x1 * sin],
                               axis=-1)

    qs = [head_norm_rope(qkv[:, h * HD:(h + 1) * HD], qw_ref[...])
          for h in range(NH)]
    ks = [head_norm_rope(qkv[:, NH * HD + h * HD:NH * HD + (h + 1) * HD],
                         kw_ref[...]) for h in range(NKV)]
    q_ref[...] = jnp.concatenate(qs, axis=-1)
    k_ref[...] = jnp.concatenate(ks, axis=-1)
    v_ref[...] = qkv[:, (NH + NKV) * HD:]


def _qkv_call(h, cos, sin, Wqkv, ln1_w, q_norm_w, k_norm_w):
    nb = T // BT
    return pl.pallas_call(
        _qkv_body,
        grid=(nb,),
        in_specs=[
            pl.BlockSpec((BT, H), lambda i: (i, 0)),
            pl.BlockSpec((BT, HD // 2), lambda i: (i, 0)),
            pl.BlockSpec((BT, HD // 2), lambda i: (i, 0)),
            pl.BlockSpec((H, (NH + 2 * NKV) * HD), lambda i: (0, 0)),
            pl.BlockSpec((1, H), lambda i: (0, 0)),
            pl.BlockSpec((1, HD), lambda i: (0, 0)),
            pl.BlockSpec((1, HD), lambda i: (0, 0)),
        ],
        out_specs=[
            pl.BlockSpec((BT, NH * HD), lambda i: (i, 0)),
            pl.BlockSpec((BT, NKV * HD), lambda i: (i, 0)),
            pl.BlockSpec((BT, NKV * HD), lambda i: (i, 0)),
        ],
        out_shape=[
            jax.ShapeDtypeStruct((T, NH * HD), jnp.float32),
            jax.ShapeDtypeStruct((T, NKV * HD), jnp.float32),
            jax.ShapeDtypeStruct((T, NKV * HD), jnp.float32),
        ],
        compiler_params=pltpu.CompilerParams(
            dimension_semantics=("arbitrary",)),
    )(h, cos, sin, Wqkv, ln1_w, q_norm_w, k_norm_w)


def _attn_body(q_ref, k_ref, v_ref, o_ref):
    i = pl.program_id(0)
    rep = NH // NKV
    scale = HD ** -0.5
    row = i * BQ + lax.broadcasted_iota(jnp.int32, (BQ, BQ), 0)
    qs = [q_ref[:, h * HD:(h + 1) * HD] * scale for h in range(NH)]

    def body(j, carry):
        ms, ls, accs = carry
        kbl = k_ref[pl.ds(j * BQ, BQ), :]
        vbl = v_ref[pl.ds(j * BQ, BQ), :]
        col = j * BQ + lax.broadcasted_iota(jnp.int32, (BQ, BQ), 1)
        causal = col <= row
        nms, nls, naccs = [], [], []
        for h in range(NH):
            kh = h // rep
            s = lax.dot_general(qs[h], kbl[:, kh * HD:(kh + 1) * HD],
                                (((1,), (1,)), ((), ())),
                                preferred_element_type=jnp.float32)
            s = jnp.where(causal, s, NEG)
            mj = jnp.max(s, axis=-1, keepdims=True)
            mn = jnp.maximum(ms[h], mj)
            alpha = jnp.exp(ms[h] - mn)
            p = jnp.exp(s - mn)
            nms.append(mn)
            nls.append(ls[h] * alpha + jnp.sum(p, axis=-1, keepdims=True))
            naccs.append(accs[h] * alpha
                         + jnp.dot(p, vbl[:, kh * HD:(kh + 1) * HD],
                                   preferred_element_type=jnp.float32))
        return nms, nls, naccs

    init = ([jnp.full((BQ, 1), NEG, jnp.float32) for _ in range(NH)],
            [jnp.zeros((BQ, 1), jnp.float32) for _ in range(NH)],
            [jnp.zeros((BQ, HD), jnp.float32) for _ in range(NH)])
    ms, ls, accs = lax.fori_loop(0, i + 1, body, init)
    o_ref[...] = jnp.concatenate(
        [accs[h] / ls[h] for h in range(NH)], axis=-1)


def _attn_call(q, k, v):
    return pl.pallas_call(
        _attn_body,
        grid=(T // BQ,),
        in_specs=[
            pl.BlockSpec((BQ, NH * HD), lambda i: (i, 0)),
            pl.BlockSpec((T, NKV * HD), lambda i: (0, 0)),
            pl.BlockSpec((T, NKV * HD), lambda i: (0, 0)),
        ],
        out_specs=pl.BlockSpec((BQ, NH * HD), lambda i: (i, 0)),
        out_shape=jax.ShapeDtypeStruct((T, NH * HD), jnp.float32),
        compiler_params=pltpu.CompilerParams(
            dimension_semantics=("arbitrary",)),
    )(q, k, v)


def _oproj_body(attn_ref, res_ref, wo_ref, ln2_ref, wg_ref,
                hs2_ref, h2_ref, wfull_ref):
    a = jnp.dot(attn_ref[...], wo_ref[...],
                preferred_element_type=jnp.float32) + res_ref[...]
    hs2_ref[...] = a
    var = jnp.mean(a * a, axis=-1, keepdims=True)
    h2 = a * lax.rsqrt(var + EPS) * ln2_ref[...]
    h2_ref[...] = h2
    logits = jnp.dot(h2, wg_ref[...], preferred_element_type=jnp.float32)
    m = jnp.max(logits, axis=-1, keepdims=True)
    p = jnp.exp(logits - m)
    p = p / jnp.sum(p, axis=-1, keepdims=True)
    ie = lax.broadcasted_iota(jnp.int32, (BT, E), 1)
    m1 = jnp.max(p, axis=-1, keepdims=True)
    i1 = jnp.min(jnp.where(p == m1, ie, E), axis=-1, keepdims=True)
    p2 = jnp.where(ie == i1, NEG, p)
    m2 = jnp.max(p2, axis=-1, keepdims=True)
    i2 = jnp.min(jnp.where(p2 == m2, ie, E), axis=-1, keepdims=True)
    denom = m1 + m2
    wfull_ref[...] = (jnp.where(ie == i1, m1 / denom, 0.0)
                      + jnp.where(ie == i2, m2 / denom, 0.0))


def _oproj_call(attn, res, Wo, ln2_w, Wg):
    nb = T // BT
    return pl.pallas_call(
        _oproj_body,
        grid=(nb,),
        in_specs=[
            pl.BlockSpec((BT, NH * HD), lambda i: (i, 0)),
            pl.BlockSpec((BT, H), lambda i: (i, 0)),
            pl.BlockSpec((NH * HD, H), lambda i: (0, 0)),
            pl.BlockSpec((1, H), lambda i: (0, 0)),
            pl.BlockSpec((H, E), lambda i: (0, 0)),
        ],
        out_specs=[
            pl.BlockSpec((BT, H), lambda i: (i, 0)),
            pl.BlockSpec((BT, H), lambda i: (i, 0)),
            pl.BlockSpec((BT, E), lambda i: (i, 0)),
        ],
        out_shape=[
            jax.ShapeDtypeStruct((T, H), jnp.float32),
            jax.ShapeDtypeStruct((T, H), jnp.float32),
            jax.ShapeDtypeStruct((T, E), jnp.float32),
        ],
        compiler_params=pltpu.CompilerParams(
            dimension_semantics=("arbitrary",)),
    )(attn, res, Wo, ln2_w, Wg)


@functools.lru_cache(maxsize=None)
def _make_sc_gather(n_rows):
    n_per = n_rows // SC_NW
    ch = min(32, n_per)
    nch = n_per // ch
    mesh = plsc.VectorSubcoreMesh(core_axis_name="c", subcore_axis_name="s")

    @functools.partial(
        pl.kernel, mesh=mesh,
        out_type=jax.ShapeDtypeStruct((n_rows, H), jnp.float32),
        scratch_types=[
            pltpu.VMEM((n_per,), jnp.int32),
            pltpu.VMEM((ch, H), jnp.float32),
            pltpu.VMEM((ch, H), jnp.float32),
            pltpu.SemaphoreType.DMA,
            pltpu.SemaphoreType.DMA,
            pltpu.SemaphoreType.DMA,
            pltpu.SemaphoreType.DMA,
        ],
    )
    def gk(table_hbm, idx_hbm, out_hbm, idx_v, rows0, rows1,
           gsem0, gsem1, osem0, osem1):
        bufs = (rows0, rows1)
        gsems = (gsem0, gsem1)
        osems = (osem0, osem1)
        wid = lax.axis_index("s") * SC_NC + lax.axis_index("c")
        base = wid * n_per
        pltpu.sync_copy(idx_hbm.at[pl.ds(base, n_per)], idx_v)
        g = [None] * nch
        o = [None] * nch
        g[0] = pltpu.async_copy(
            table_hbm.at[idx_v.at[pl.ds(0, ch)]], bufs[0], gsems[0])
        for c in range(nch):
            g[c].wait()
            if c + 1 < nch:
                if c >= 1:
                    o[c - 1].wait()
                b = (c + 1) % 2
                g[c + 1] = pltpu.async_copy(
                    table_hbm.at[idx_v.at[pl.ds((c + 1) * ch, ch)]],
                    bufs[b], gsems[b])
            o[c] = pltpu.async_copy(
                bufs[c % 2], out_hbm.at[pl.ds(base + c * ch, ch)],
                osems[c % 2])
        if nch >= 2:
            o[nch - 2].wait()
        o[nch - 1].wait()

    return gk


def _sc_gather_rows(table, idx):
    return _make_sc_gather(idx.shape[0])(table, idx)


def _moe_body(te_ref, x_ref, wgu_ref, wd_ref, y_ref):
    x = x_ref[...]
    gu = jnp.dot(x, wgu_ref[0], preferred_element_type=jnp.float32)
    g = gu[:, :FF]
    u = gu[:, FF:]
    act = g * jax.nn.sigmoid(g) * u
    y_ref[...] = jnp.dot(act, wd_ref[0], preferred_element_type=jnp.float32)


def _moe_call(x_sorted, tile_expert, W_gateup, W_down):
    grid_spec = pltpu.PrefetchScalarGridSpec(
        num_scalar_prefetch=1,
        grid=(NT,),
        in_specs=[
            pl.BlockSpec((TILE, H), lambda g, te: (g, 0)),
            pl.BlockSpec((1, H, 2 * FF), lambda g, te: (te[g], 0, 0)),
            pl.BlockSpec((1, FF, H), lambda g, te: (te[g], 0, 0)),
        ],
        out_specs=pl.BlockSpec((TILE, H), lambda g, te: (g, 0)),
    )
    return pl.pallas_call(
        _moe_body,
        grid_spec=grid_spec,
        out_shape=jax.ShapeDtypeStruct((PADMAX, H), jnp.float32),
        compiler_params=pltpu.CompilerParams(
            dimension_semantics=("arbitrary",)),
    )(tile_expert, x_sorted, W_gateup, W_down)


def _combine_body(res_ref, g0_ref, g1_ref, w0_ref, w1_ref, out_ref):
    out_ref[...] = (res_ref[...]
                    + g0_ref[...] * w0_ref[:, :1]
                    + g1_ref[...] * w1_ref[:, :1])


def _combine_call(res, g01, w0b, w1b):
    nb = T // BT
    return pl.pallas_call(
        _combine_body,
        grid=(nb,),
        in_specs=[
            pl.BlockSpec((BT, H), lambda i: (i, 0)),
            pl.BlockSpec((BT, H), lambda i: (i, 0)),
            pl.BlockSpec((BT, H), lambda i: (i + T // BT, 0)),
            pl.BlockSpec((BT, 128), lambda i: (i, 0)),
            pl.BlockSpec((BT, 128), lambda i: (i, 0)),
        ],
        out_specs=pl.BlockSpec((BT, H), lambda i: (i, 0)),
        out_shape=jax.ShapeDtypeStruct((T, H), jnp.float32),
        compiler_params=pltpu.CompilerParams(
            dimension_semantics=("arbitrary",)),
    )(res, g01, g01, w0b, w1b)


def kernel(hidden_states, positions, Wqkv, Wo, q_norm_w, k_norm_w,
           ln1_w, ln2_w, Wg, W_gateup, W_down):
    f32 = jnp.float32
    inv_freq = 1.0 / (THETA ** (np.arange(0, HD, 2, dtype=np.float32) / HD))
    freqs = positions.astype(f32)[:, None] * inv_freq[None, :]
    cos = jnp.cos(freqs)
    sin = jnp.sin(freqs)

    q, k, v = _qkv_call(hidden_states, cos, sin, Wqkv,
                        ln1_w.reshape(1, H),
                        q_norm_w.reshape(1, HD), k_norm_w.reshape(1, HD))
    attn = _attn_call(q, k, v)
    hs2, h2, wfull = _oproj_call(attn, hidden_states, Wo,
                                 ln2_w.reshape(1, H), Wg)

    cnt = (wfull > 0.0).astype(jnp.int32)
    csum = jnp.cumsum(cnt, axis=0)
    prefix = csum - cnt
    counts = csum[-1]
    pcounts = ((counts + TILE - 1) // TILE) * TILE
    pend = jnp.cumsum(pcounts)
    poff = pend - pcounts
    pos = poff[None, :] + prefix
    dest = jnp.where(cnt > 0, pos, PADMAX)
    tok = jnp.broadcast_to(jnp.arange(T, dtype=jnp.int32)[:, None], (T, E))
    gather_idx = jnp.zeros((PADMAX,), jnp.int32).at[dest.reshape(-1)].set(
        tok.reshape(-1), mode="drop")
    tile_expert = jnp.minimum(
        jnp.searchsorted(pend, jnp.arange(NT, dtype=jnp.int32) * TILE,
                         side="right"),
        E - 1).astype(jnp.int32)
    posm = jnp.where(cnt > 0, pos, PADMAX - 1)
    order = jnp.argsort(posm, axis=1)[:, :TOPK]
    pos01 = jnp.take_along_axis(posm, order, axis=1).astype(jnp.int32)
    w01 = jnp.take_along_axis(wfull, order, axis=1)
    poscat = jnp.concatenate([pos01[:, 0], pos01[:, 1]])
    w0b = jnp.broadcast_to(w01[:, 0:1], (T, 128))
    w1b = jnp.broadcast_to(w01[:, 1:2], (T, 128))

    x_sorted = _sc_gather_rows(h2, gather_idx)
    y_sorted = _moe_call(x_sorted, tile_expert, W_gateup, W_down)
    g01 = _sc_gather_rows(y_sorted, poscat)
    return _combine_call(hs2, g01, w0b, w1b)


# speedup vs baseline: 1.1706x; 1.1706x over previous
import functools

import jax
import jax.numpy as jnp
import numpy as np
from jax import lax
from jax.experimental import pallas as pl
from jax.experimental.pallas import tpu as pltpu
from jax.experimental.pallas import tpu_sc as plsc

T = 2048
H = 1024
NH = 16
NKV = 8
HD = 64
E = 8
TOPK = 2
FF = 512
EPS = 1e-6
THETA = 10000.0

BT = 256
BQ = 256
TILE = 128
PADMAX = TOPK * T + E * TILE
NT = PADMAX // TILE
NEG = -1e30

SC_NC = 2
SC_NS = 16
SC_NW = SC_NC * SC_NS


def _qkv_body(x_ref, cos_ref, sin_ref, wqkv_ref, ln1_ref, qw_ref, kw_ref,
              q_ref, k_ref, v_ref):
    x = x_ref[...]
    var = jnp.mean(x * x, axis=-1, keepdims=True)
    xn = x * lax.rsqrt(var + EPS) * ln1_ref[...]
    qkv = jnp.dot(xn, wqkv_ref[...], preferred_element_type=jnp.float32)
    cos = cos_ref[...]
    sin = sin_ref[...]

    def head_norm_rope(xh, w):
        ss = jnp.mean(xh * xh, axis=-1, keepdims=True)
        xh = xh * lax.rsqrt(ss + EPS) * w
        x1 = xh[:, :HD // 2]
        x2 = xh[:, HD // 2:]
        return jnp.concatenate([x1 * cos - x2 * sin, x2 * cos + x1 * sin],
                               axis=-1)

    qs = [head_norm_rope(qkv[:, h * HD:(h + 1) * HD], qw_ref[...])
          for h in range(NH)]
    ks = [head_norm_rope(qkv[:, NH * HD + h * HD:NH * HD + (h + 1) * HD],
                         kw_ref[...]) for h in range(NKV)]
    q_ref[...] = jnp.concatenate(qs, axis=-1)
    k_ref[...] = jnp.concatenate(ks, axis=-1)
    v_ref[...] = qkv[:, (NH + NKV) * HD:]


def _qkv_call(h, cos, sin, Wqkv, ln1_w, q_norm_w, k_norm_w):
    nb = T // BT
    return pl.pallas_call(
        _qkv_body,
        grid=(nb,),
        in_specs=[
            pl.BlockSpec((BT, H), lambda i: (i, 0)),
            pl.BlockSpec((BT, HD // 2), lambda i: (i, 0)),
            pl.BlockSpec((BT, HD // 2), lambda i: (i, 0)),
            pl.BlockSpec((H, (NH + 2 * NKV) * HD), lambda i: (0, 0)),
            pl.BlockSpec((1, H), lambda i: (0, 0)),
            pl.BlockSpec((1, HD), lambda i: (0, 0)),
            pl.BlockSpec((1, HD), lambda i: (0, 0)),
        ],
        out_specs=[
            pl.BlockSpec((BT, NH * HD), lambda i: (i, 0)),
            pl.BlockSpec((BT, NKV * HD), lambda i: (i, 0)),
            pl.BlockSpec((BT, NKV * HD), lambda i: (i, 0)),
        ],
        out_shape=[
            jax.ShapeDtypeStruct((T, NH * HD), jnp.float32),
            jax.ShapeDtypeStruct((T, NKV * HD), jnp.float32),
            jax.ShapeDtypeStruct((T, NKV * HD), jnp.float32),
        ],
        compiler_params=pltpu.CompilerParams(
            dimension_semantics=("arbitrary",)),
    )(h, cos, sin, Wqkv, ln1_w, q_norm_w, k_norm_w)


def _attn_body(q_ref, k_ref, v_ref, o_ref):
    i = pl.program_id(0)
    rep = NH // NKV
    scale = HD ** -0.5
    row = i * BQ + lax.broadcasted_iota(jnp.int32, (BQ, T), 0)
    col = lax.broadcasted_iota(jnp.int32, (BQ, T), 1)
    causal = col <= row
    os = []
    for h in range(NH):
        kh = h // rep
        q = q_ref[:, h * HD:(h + 1) * HD] * scale
        k = k_ref[:, kh * HD:(kh + 1) * HD]
        s = lax.dot_general(q, k, (((1,), (1,)), ((), ())),
                            preferred_element_type=jnp.float32)
        s = jnp.where(causal, s, NEG)
        m = jnp.max(s, axis=-1, keepdims=True)
        p = jnp.exp(s - m)
        l = jnp.sum(p, axis=-1, keepdims=True)
        v = v_ref[:, kh * HD:(kh + 1) * HD]
        os.append(jnp.dot(p, v, preferred_element_type=jnp.float32) / l)
    o_ref[...] = jnp.concatenate(os, axis=-1)


def _attn_call(q, k, v):
    return pl.pallas_call(
        _attn_body,
        grid=(T // BQ,),
        in_specs=[
            pl.BlockSpec((BQ, NH * HD), lambda i: (i, 0)),
            pl.BlockSpec((T, NKV * HD), lambda i: (0, 0)),
            pl.BlockSpec((T, NKV * HD), lambda i: (0, 0)),
        ],
        out_specs=pl.BlockSpec((BQ, NH * HD), lambda i: (i, 0)),
        out_shape=jax.ShapeDtypeStruct((T, NH * HD), jnp.float32),
        compiler_params=pltpu.CompilerParams(
            dimension_semantics=("arbitrary",)),
    )(q, k, v)


def _oproj_body(attn_ref, res_ref, wo_ref, ln2_ref, wg_ref,
                hs2_ref, h2_ref, wfull_ref):
    a = jnp.dot(attn_ref[...], wo_ref[...],
                preferred_element_type=jnp.float32) + res_ref[...]
    hs2_ref[...] = a
    var = jnp.mean(a * a, axis=-1, keepdims=True)
    h2 = a * lax.rsqrt(var + EPS) * ln2_ref[...]
    h2_ref[...] = h2
    logits = jnp.dot(h2, wg_ref[...], preferred_element_type=jnp.float32)
    m = jnp.max(logits, axis=-1, keepdims=True)
    p = jnp.exp(logits - m)
    p = p / jnp.sum(p, axis=-1, keepdims=True)
    ie = lax.broadcasted_iota(jnp.int32, (BT, E), 1)
    m1 = jnp.max(p, axis=-1, keepdims=True)
    i1 = jnp.min(jnp.where(p == m1, ie, E), axis=-1, keepdims=True)
    p2 = jnp.where(ie == i1, NEG, p)
    m2 = jnp.max(p2, axis=-1, keepdims=True)
    i2 = jnp.min(jnp.where(p2 == m2, ie, E), axis=-1, keepdims=True)
    denom = m1 + m2
    wfull_ref[...] = (jnp.where(ie == i1, m1 / denom, 0.0)
                      + jnp.where(ie == i2, m2 / denom, 0.0))


def _oproj_call(attn, res, Wo, ln2_w, Wg):
    nb = T // BT
    return pl.pallas_call(
        _oproj_body,
        grid=(nb,),
        in_specs=[
            pl.BlockSpec((BT, NH * HD), lambda i: (i, 0)),
            pl.BlockSpec((BT, H), lambda i: (i, 0)),
            pl.BlockSpec((NH * HD, H), lambda i: (0, 0)),
            pl.BlockSpec((1, H), lambda i: (0, 0)),
            pl.BlockSpec((H, E), lambda i: (0, 0)),
        ],
        out_specs=[
            pl.BlockSpec((BT, H), lambda i: (i, 0)),
            pl.BlockSpec((BT, H), lambda i: (i, 0)),
            pl.BlockSpec((BT, E), lambda i: (i, 0)),
        ],
        out_shape=[
            jax.ShapeDtypeStruct((T, H), jnp.float32),
            jax.ShapeDtypeStruct((T, H), jnp.float32),
            jax.ShapeDtypeStruct((T, E), jnp.float32),
        ],
        compiler_params=pltpu.CompilerParams(
            dimension_semantics=("arbitrary",)),
    )(attn, res, Wo, ln2_w, Wg)


@functools.lru_cache(maxsize=None)
def _make_sc_gather(n_rows):
    n_per = n_rows // SC_NW
    ch = min(32, n_per)
    nch = n_per // ch
    mesh = plsc.VectorSubcoreMesh(core_axis_name="c", subcore_axis_name="s")

    @functools.partial(
        pl.kernel, mesh=mesh,
        out_type=jax.ShapeDtypeStruct((n_rows, H), jnp.float32),
        scratch_types=[
            pltpu.VMEM((n_per,), jnp.int32),
            pltpu.VMEM((ch, H), jnp.float32),
            pltpu.VMEM((ch, H), jnp.float32),
            pltpu.SemaphoreType.DMA,
            pltpu.SemaphoreType.DMA,
            pltpu.SemaphoreType.DMA,
            pltpu.SemaphoreType.DMA,
        ],
    )
    def gk(table_hbm, idx_hbm, out_hbm, idx_v, rows0, rows1,
           gsem0, gsem1, osem0, osem1):
        bufs = (rows0, rows1)
        gsems = (gsem0, gsem1)
        osems = (osem0, osem1)
        wid = lax.axis_index("s") * SC_NC + lax.axis_index("c")
        base = wid * n_per
        pltpu.sync_copy(idx_hbm.at[pl.ds(base, n_per)], idx_v)
        g = [None] * nch
        o = [None] * nch
        g[0] = pltpu.async_copy(
            table_hbm.at[idx_v.at[pl.ds(0, ch)]], bufs[0], gsems[0])
        for c in range(nch):
            g[c].wait()
            if c + 1 < nch:
                if c >= 1:
                    o[c - 1].wait()
                b = (c + 1) % 2
                g[c + 1] = pltpu.async_copy(
                    table_hbm.at[idx_v.at[pl.ds((c + 1) * ch, ch)]],
                    bufs[b], gsems[b])
            o[c] = pltpu.async_copy(
                bufs[c % 2], out_hbm.at[pl.ds(base + c * ch, ch)],
                osems[c % 2])
        if nch >= 2:
            o[nch - 2].wait()
        o[nch - 1].wait()

    return gk


def _sc_gather_rows(table, idx):
    return _make_sc_gather(idx.shape[0])(table, idx)


def _moe_body(te_ref, x_ref, wgu_ref, wd_ref, y_ref):
    x = x_ref[...]
    gu = jnp.dot(x, wgu_ref[0], preferred_element_type=jnp.float32)
    g = gu[:, :FF]
    u = gu[:, FF:]
    act = g * jax.nn.sigmoid(g) * u
    y_ref[...] = jnp.dot(act, wd_ref[0], preferred_element_type=jnp.float32)


def _moe_call(x_sorted, tile_expert, W_gateup, W_down):
    grid_spec = pltpu.PrefetchScalarGridSpec(
        num_scalar_prefetch=1,
        grid=(NT,),
        in_specs=[
            pl.BlockSpec((TILE, H), lambda g, te: (g, 0)),
            pl.BlockSpec((1, H, 2 * FF), lambda g, te: (te[g], 0, 0)),
            pl.BlockSpec((1, FF, H), lambda g, te: (te[g], 0, 0)),
        ],
        out_specs=pl.BlockSpec((TILE, H), lambda g, te: (g, 0)),
    )
    return pl.pallas_call(
        _moe_body,
        grid_spec=grid_spec,
        out_shape=jax.ShapeDtypeStruct((PADMAX, H), jnp.float32),
        compiler_params=pltpu.CompilerParams(
            dimension_semantics=("arbitrary",)),
    )(tile_expert, x_sorted, W_gateup, W_down)


def _combine_body(res_ref, g0_ref, g1_ref, w0_ref, w1_ref, out_ref):
    out_ref[...] = (res_ref[...]
                    + g0_ref[...] * w0_ref[:, :1]
                    + g1_ref[...] * w1_ref[:, :1])


def _combine_call(res, g01, w0b, w1b):
    nb = T // BT
    return pl.pallas_call(
        _combine_body,
        grid=(nb,),
        in_specs=[
            pl.BlockSpec((BT, H), lambda i: (i, 0)),
            pl.BlockSpec((BT, H), lambda i: (i, 0)),
            pl.BlockSpec((BT, H), lambda i: (i + T // BT, 0)),
            pl.BlockSpec((BT, 128), lambda i: (i, 0)),
            pl.BlockSpec((BT, 128), lambda i: (i, 0)),
        ],
        out_specs=pl.BlockSpec((BT, H), lambda i: (i, 0)),
        out_shape=jax.ShapeDtypeStruct((T, H), jnp.float32),
        compiler_params=pltpu.CompilerParams(
            dimension_semantics=("arbitrary",)),
    )(res, g01, g01, w0b, w1b)


def kernel(hidden_states, positions, Wqkv, Wo, q_norm_w, k_norm_w,
           ln1_w, ln2_w, Wg, W_gateup, W_down):
    f32 = jnp.float32
    inv_freq = 1.0 / (THETA ** (np.arange(0, HD, 2, dtype=np.float32) / HD))
    freqs = positions.astype(f32)[:, None] * inv_freq[None, :]
    cos = jnp.cos(freqs)
    sin = jnp.sin(freqs)

    q, k, v = _qkv_call(hidden_states, cos, sin, Wqkv,
                        ln1_w.reshape(1, H),
                        q_norm_w.reshape(1, HD), k_norm_w.reshape(1, HD))
    attn = _attn_call(q, k, v)
    hs2, h2, wfull = _oproj_call(attn, hidden_states, Wo,
                                 ln2_w.reshape(1, H), Wg)

    cnt = (wfull > 0.0).astype(jnp.int32)
    csum = jnp.cumsum(cnt, axis=0)
    prefix = csum - cnt
    counts = csum[-1]
    pcounts = ((counts + TILE - 1) // TILE) * TILE
    pend = jnp.cumsum(pcounts)
    poff = pend - pcounts
    pos = poff[None, :] + prefix
    dest = jnp.where(cnt > 0, pos, PADMAX)
    tok = jnp.broadcast_to(jnp.arange(T, dtype=jnp.int32)[:, None], (T, E))
    gather_idx = jnp.zeros((PADMAX,), jnp.int32).at[dest.reshape(-1)].set(
        tok.reshape(-1), mode="drop")
    tile_expert = jnp.minimum(
        jnp.searchsorted(pend, jnp.arange(NT, dtype=jnp.int32) * TILE,
                         side="right"),
        E - 1).astype(jnp.int32)
    posm = jnp.where(cnt > 0, pos, PADMAX - 1)
    order = jnp.argsort(posm, axis=1)[:, :TOPK]
    pos01 = jnp.take_along_axis(posm, order, axis=1).astype(jnp.int32)
    w01 = jnp.take_along_axis(wfull, order, axis=1)
    poscat = jnp.concatenate([pos01[:, 0], pos01[:, 1]])
    w0b = jnp.broadcast_to(w01[:, 0:1], (T, 128))
    w1b = jnp.broadcast_to(w01[:, 1:2], (T, 128))

    x_sorted = _sc_gather_rows(h2, gather_idx)
    y_sorted = _moe_call(x_sorted, tile_expert, W_gateup, W_down)
    g01 = _sc_gather_rows(y_sorted, poscat)
    return _combine_call(hs2, g01, w0b, w1b)
